# Initial kernel scaffold; baseline (speedup 1.0000x reference)
import jax
import jax.numpy as jnp
from jax.experimental import pallas as pl


def kernel(input, grid):
    raise NotImplementedError("write your pallas kernel here")


# speedup vs baseline: 4.3106x; 4.3106x over previous
import jax
import jax.numpy as jnp
from jax import lax
from jax.experimental import pallas as pl
from jax.experimental.pallas import tpu as pltpu
from jax.experimental.pallas import tpu_sc as plsc

_N, _C, _L = 32, 128, 8192
_LANES = 16
_CHUNKS = _L // _LANES


def _sc_body(inp_hbm, grid_hbm, out_hbm, grid_v, idx_v, w1_v, in_v, out_v):
    core = lax.axis_index("c")
    sub = lax.axis_index("s")
    w = sub * 2 + core

    pltpu.sync_copy(grid_hbm.at[w], grid_v)

    def _widx(k, carry):
        g = grid_v[pl.ds(k * _LANES, _LANES)]
        ix = (g + 1.0) * (0.5 * (_L - 1))
        ix = jnp.minimum(jnp.maximum(ix, 0.0), float(_L - 1))
        i0 = ix.astype(jnp.int32)
        w1 = ix - i0.astype(jnp.float32)
        idx_v[pl.ds(k * _LANES, _LANES)] = i0
        w1_v[pl.ds(k * _LANES, _LANES)] = w1
        return carry

    lax.fori_loop(0, _CHUNKS, _widx, 0)

    def _chan(c, carry):
        pltpu.sync_copy(inp_hbm.at[w, c], in_v)

        def _chunk(k, inner):
            i0 = idx_v[pl.ds(k * _LANES, _LANES)]
            w1 = w1_v[pl.ds(k * _LANES, _LANES)]
            i1 = jnp.minimum(i0 + 1, _L - 1)
            v0 = plsc.load_gather(in_v, [i0])
            v1 = plsc.load_gather(in_v, [i1])
            out_v[pl.ds(k * _LANES, _LANES)] = v0 + w1 * (v1 - v0)
            return inner

        lax.fori_loop(0, _CHUNKS, _chunk, 0)
        pltpu.sync_copy(out_v, out_hbm.at[w, c])
        return carry

    lax.fori_loop(0, _C, _chan, 0)


@jax.jit
def kernel(input, grid):
    mesh = plsc.VectorSubcoreMesh(core_axis_name="c", subcore_axis_name="s")
    f = pl.kernel(
        _sc_body,
        mesh=mesh,
        out_type=jax.ShapeDtypeStruct((_N, _C, _L), jnp.float32),
        compiler_params=pltpu.CompilerParams(needs_layout_passes=False),
        scratch_types=[
            pltpu.VMEM((_L,), jnp.float32),
            pltpu.VMEM((_L,), jnp.int32),
            pltpu.VMEM((_L,), jnp.float32),
            pltpu.VMEM((_L,), jnp.float32),
            pltpu.VMEM((_L,), jnp.float32),
        ],
    )
    return f(input, grid)
